# baseline (speedup 1.0000x reference)
import functools

import jax
import jax.numpy as jnp
from jax import lax
from jax.experimental import pallas as pl
from jax.experimental.pallas import tpu as pltpu
from jax.experimental.pallas import tpu_sc as plsc


def _table_body(emb_ref, w1_ref, b1_ref, w2_ref, b2_ref, out_ref):
    e = emb_ref[...]
    h = lax.dot_general(e, w1_ref[...], (((1,), (1,)), ((), ())),
                        preferred_element_type=jnp.float32)
    h = jnp.maximum(h + b1_ref[...], 0.0)
    logit = lax.dot_general(h, w2_ref[...], (((1,), (1,)), ((), ())),
                            preferred_element_type=jnp.float32)
    out_ref[...] = jax.nn.sigmoid(logit + b2_ref[0, 0])


@functools.lru_cache(maxsize=None)
def _make_table_fn(vocab, hid, vblk):
    nblk = vocab // vblk
    hid2 = hid // 2
    return pl.pallas_call(
        _table_body,
        grid=(nblk,),
        in_specs=[
            pl.BlockSpec((vblk, hid), lambda i: (i, 0)),
            pl.BlockSpec((hid2, hid), lambda i: (0, 0)),
            pl.BlockSpec((1, hid2), lambda i: (0, 0)),
            pl.BlockSpec((8, hid2), lambda i: (0, 0)),
            pl.BlockSpec(memory_space=pltpu.SMEM),
        ],
        out_specs=pl.BlockSpec((vblk, 8), lambda i: (i, 0)),
        out_shape=jax.ShapeDtypeStruct((vocab, 8), jnp.float32),
        compiler_params=pltpu.CompilerParams(
            dimension_semantics=("arbitrary",)),
    )


@functools.lru_cache(maxsize=None)
def _make_gather_mean(vocab, batch, seq):
    info = plsc.get_sparse_core_info()
    nc, ns, lanes = info.num_cores, info.num_subcores, info.num_lanes
    nw = nc * ns
    assert batch % nw == 0
    rows_w = batch // nw
    idx_w = rows_w * seq
    nfull = seq // lanes
    rem = seq % lanes
    mesh = plsc.VectorSubcoreMesh(core_axis_name="c", subcore_axis_name="s")

    @functools.partial(
        pl.kernel,
        mesh=mesh,
        out_type=jax.ShapeDtypeStruct((batch,), jnp.float32),
        scratch_types=[
            pltpu.VMEM((idx_w,), jnp.int32),
            pltpu.VMEM((idx_w,), jnp.float32),
            pltpu.VMEM((rows_w,), jnp.float32),
            pltpu.SemaphoreType.DMA,
        ],
    )
    def gather_mean(table_hbm, ids_hbm, out_hbm, idx_v, vals_v, res_v, sem):
        wid = lax.axis_index("s") * nc + lax.axis_index("c")
        base = wid * idx_w
        pltpu.sync_copy(ids_hbm.at[pl.ds(base, idx_w)], idx_v)
        pltpu.async_copy(table_hbm.at[idx_v], vals_v, sem).wait()

        lane = lax.iota(jnp.int32, lanes)
        tail_mask = lane >= (lanes - rem)
        inv_s = jnp.float32(1.0 / seq)
        dnums = lax.GatherDimensionNumbers(
            offset_dims=(), collapsed_slice_dims=(0,), start_index_map=(0,))

        def permute(x, idx):
            return lax.gather(
                x, idx[:, None], dnums, slice_sizes=(1,),
                mode=lax.GatherScatterMode.PROMISE_IN_BOUNDS)

        def hsum_splat(x):
            for sh in (8, 4, 2, 1):
                x = x + permute(x, lane ^ sh)
            return x

        ngrp = rows_w // lanes

        def grp(g, carry):
            gb = pl.multiple_of(g * (lanes * seq), 8)
            out_vec = jnp.zeros((lanes,), jnp.float32)
            for l in range(lanes):
                b = gb + l * seq
                acc = vals_v[pl.ds(b, lanes)]
                for j in range(1, nfull):
                    acc = acc + vals_v[pl.ds(b + j * lanes, lanes)]
                if rem:
                    tail = vals_v[pl.ds(b + seq - lanes, lanes)]
                    acc = acc + jnp.where(tail_mask, tail, 0.0)
                out_vec = jnp.where(lane == l, hsum_splat(acc) * inv_s,
                                    out_vec)
            res_v[pl.ds(pl.multiple_of(g * lanes, 8), lanes)] = out_vec
            return carry

        lax.fori_loop(0, ngrp, grp, 0)
        pltpu.sync_copy(res_v, out_hbm.at[pl.ds(wid * rows_w, rows_w)])

    return gather_mean


def kernel(input_ids, embed_table, W1, b1, W2, b2):
    batch, seq = input_ids.shape
    vocab, hid = embed_table.shape
    vblk = 8000
    table_fn = _make_table_fn(vocab, hid, vblk)
    w2t = jnp.tile(W2.reshape(1, -1), (8, 1))
    diff = table_fn(embed_table, W1, b1.reshape(1, -1), w2t, b2.reshape(1, 1))
    diff = diff.reshape(vocab * 8)
    ids = input_ids.reshape(batch * seq).astype(jnp.int32) * 8
    gather_fn = _make_gather_mean(vocab, batch, seq)
    return gather_fn(diff, ids)


# speedup vs baseline: 1.7158x; 1.7158x over previous
import functools

import jax
import jax.numpy as jnp
from jax import lax
from jax.experimental import pallas as pl
from jax.experimental.pallas import tpu as pltpu
from jax.experimental.pallas import tpu_sc as plsc


def _table_body(emb_ref, w1_ref, b1_ref, w2_ref, b2_ref, out_ref):
    e = emb_ref[...]
    h = lax.dot_general(e, w1_ref[...], (((1,), (1,)), ((), ())),
                        preferred_element_type=jnp.float32)
    h = jnp.maximum(h + b1_ref[...], 0.0)
    logit = lax.dot_general(w2_ref[...], h, (((1,), (1,)), ((), ())),
                            preferred_element_type=jnp.float32)
    out_ref[...] = jax.nn.sigmoid(logit + b2_ref[0, 0]).reshape(logit.shape[1])


@functools.lru_cache(maxsize=None)
def _make_table_fn(vocab, hid, vblk):
    nblk = -(-vocab // vblk)
    hid2 = hid // 2
    return pl.pallas_call(
        _table_body,
        grid=(nblk,),
        in_specs=[
            pl.BlockSpec((vblk, hid), lambda i: (i, 0)),
            pl.BlockSpec((hid2, hid), lambda i: (0, 0)),
            pl.BlockSpec((1, hid2), lambda i: (0, 0)),
            pl.BlockSpec((1, hid2), lambda i: (0, 0)),
            pl.BlockSpec(memory_space=pltpu.SMEM),
        ],
        out_specs=pl.BlockSpec((vblk,), lambda i: (i,)),
        out_shape=jax.ShapeDtypeStruct((vocab,), jnp.float32),
        compiler_params=pltpu.CompilerParams(
            dimension_semantics=("arbitrary",)),
    )


@functools.lru_cache(maxsize=None)
def _make_gather_mean(vocab, batch, seq):
    info = plsc.get_sparse_core_info()
    nc, ns, lanes = info.num_cores, info.num_subcores, info.num_lanes
    nw = nc * ns
    assert batch % nw == 0
    rows_w = batch // nw
    idx_w = rows_w * seq
    nfull = seq // lanes
    rem = seq % lanes
    mesh = plsc.VectorSubcoreMesh(core_axis_name="c", subcore_axis_name="s")

    @functools.partial(
        pl.kernel,
        mesh=mesh,
        out_type=jax.ShapeDtypeStruct((batch,), jnp.float32),
        scratch_types=[
            pltpu.VMEM((idx_w,), jnp.int32),
            pltpu.VMEM((idx_w,), jnp.float32),
            pltpu.VMEM((rows_w,), jnp.float32),
            pltpu.SemaphoreType.DMA,
        ],
    )
    def gather_mean(table_hbm, ids_hbm, out_hbm, idx_v, vals_v, res_v, sem):
        wid = lax.axis_index("s") * nc + lax.axis_index("c")
        base = wid * idx_w
        pltpu.sync_copy(ids_hbm.at[pl.ds(base, idx_w)], idx_v)
        pltpu.async_copy(table_hbm.at[idx_v], vals_v, sem).wait()

        lane = lax.iota(jnp.int32, lanes)
        tail_mask = lane >= (lanes - rem)
        inv_s = jnp.float32(1.0 / seq)
        dnums = lax.GatherDimensionNumbers(
            offset_dims=(), collapsed_slice_dims=(0,), start_index_map=(0,))

        def permute(x, idx):
            return lax.gather(
                x, idx[:, None], dnums, slice_sizes=(1,),
                mode=lax.GatherScatterMode.PROMISE_IN_BOUNDS)

        def hsum_splat(x):
            for sh in (8, 4, 2, 1):
                x = x + permute(x, lane ^ sh)
            return x

        ngrp = rows_w // lanes

        def grp(g, carry):
            gb = pl.multiple_of(g * (lanes * seq), 8)
            out_vec = jnp.zeros((lanes,), jnp.float32)
            for l in range(lanes):
                b = gb + l * seq
                acc = vals_v[pl.ds(b, lanes)]
                for j in range(1, nfull):
                    acc = acc + vals_v[pl.ds(b + j * lanes, lanes)]
                if rem:
                    tail = vals_v[pl.ds(b + seq - lanes, lanes)]
                    acc = acc + jnp.where(tail_mask, tail, 0.0)
                out_vec = jnp.where(lane == l, hsum_splat(acc) * inv_s,
                                    out_vec)
            res_v[pl.ds(pl.multiple_of(g * lanes, 8), lanes)] = out_vec
            return carry

        lax.fori_loop(0, ngrp, grp, 0)
        pltpu.sync_copy(res_v, out_hbm.at[pl.ds(wid * rows_w, rows_w)])

    return gather_mean


def kernel(input_ids, embed_table, W1, b1, W2, b2):
    batch, seq = input_ids.shape
    vocab, hid = embed_table.shape
    vblk = 8192
    table_fn = _make_table_fn(vocab, hid, vblk)
    diff = table_fn(embed_table, W1, b1.reshape(1, -1), W2.reshape(1, -1),
                    b2.reshape(1, 1))
    ids = input_ids.reshape(batch * seq).astype(jnp.int32)
    gather_fn = _make_gather_mean(vocab, batch, seq)
    return gather_fn(diff, ids)
